# Initial kernel scaffold; baseline (speedup 1.0000x reference)
import jax
import jax.numpy as jnp
from jax.experimental import pallas as pl


def kernel(x, rel_table):
    raise NotImplementedError("write your pallas kernel here")


# speedup vs baseline: 225.0527x; 225.0527x over previous
import jax
import jax.numpy as jnp
from jax.experimental import pallas as pl

_MAX_REL = 32
_NIDX = 2 * _MAX_REL + 1
_KPAD = 128


def _body(x_ref, tab_ref, o_ref, *, seq_len, block_s):
    s0 = pl.program_id(1) * block_s
    j = s0 + jax.lax.broadcasted_iota(jnp.int32, (block_s, _KPAD), 0)
    k = jax.lax.broadcasted_iota(jnp.int32, (block_s, _KPAD), 1)
    d = k - _MAX_REL
    mid = ((j - d >= 0) & (j - d <= seq_len - 1) & (k >= 1) & (k <= _NIDX - 2))
    w = jnp.where(
        k == 0,
        jnp.maximum(0, (seq_len - _MAX_REL) - j),
        jnp.where(k == _NIDX - 1, jnp.maximum(0, j - (_MAX_REL - 1)),
                  mid.astype(jnp.int32)),
    ).astype(jnp.float32)
    w = jnp.where(k >= _NIDX, 0.0, w)
    bias = jax.lax.dot(w, tab_ref[...],
                       preferred_element_type=jnp.float32) * (1.0 / seq_len)
    o_ref[...] = x_ref[...] + bias[None, :, :]


def kernel(x, rel_table):
    batch, seq_len, hidden = x.shape
    block_s = 512
    tab = jnp.zeros((_KPAD, hidden), rel_table.dtype).at[:_NIDX].set(rel_table)
    grid = (batch, seq_len // block_s)
    return pl.pallas_call(
        lambda xr, tr, orr: _body(xr, tr, orr, seq_len=seq_len,
                                  block_s=block_s),
        grid=grid,
        in_specs=[
            pl.BlockSpec((1, block_s, hidden), lambda b, s: (b, s, 0)),
            pl.BlockSpec((_KPAD, hidden), lambda b, s: (0, 0)),
        ],
        out_specs=pl.BlockSpec((1, block_s, hidden), lambda b, s: (b, s, 0)),
        out_shape=jax.ShapeDtypeStruct(x.shape, x.dtype),
    )(x, tab)
